# Initial kernel scaffold; baseline (speedup 1.0000x reference)
import functools

import jax
import jax.numpy as jnp
from jax import lax
from jax.experimental import pallas as pl
from jax.experimental.pallas import tpu as pltpu
from jax.experimental.pallas import tpu_sc as plsc

NC = 2
NS = 16
L = 16
NW = NC * NS

B = 16384
TAB = 1024
BPW = B // NW
VECS = BPW // L


def _softplus_floor(s):
    e = jnp.exp(-jnp.abs(s))
    u = e / (e + 2.0)
    t = u * u
    p = 1.0 / 11.0
    p = p * t + 1.0 / 9.0
    p = p * t + 1.0 / 7.0
    p = p * t + 1.0 / 5.0
    p = p * t + 1.0 / 3.0
    p = p * t + 1.0
    log1p_e = 2.0 * u * p
    sp = jnp.maximum(s, 0.0) + log1p_e
    return jnp.maximum(sp, 0.001)


@functools.partial(
    pl.kernel,
    out_type=(
        jax.ShapeDtypeStruct((B,), jnp.float32),
        jax.ShapeDtypeStruct((B,), jnp.float32),
    ),
    mesh=plsc.VectorSubcoreMesh(
        core_axis_name="c", subcore_axis_name="s", num_cores=NC, num_subcores=NS
    ),
    scratch_types=[
        pltpu.VMEM((TAB,), jnp.float32),
        pltpu.VMEM((TAB,), jnp.float32),
        pltpu.VMEM((BPW,), jnp.int32),
        pltpu.VMEM((BPW,), jnp.float32),
        pltpu.VMEM((BPW,), jnp.float32),
    ],
)
def _gather_softplus(loc_hbm, scale_hbm, idx_hbm, out_loc, out_scale,
                     loc_tab, scale_tab, idx_v, oloc_v, oscale_v):
    wid = lax.axis_index("s") * NC + lax.axis_index("c")
    base = wid * BPW
    pltpu.sync_copy(idx_hbm.at[pl.ds(base, BPW)], idx_v)
    pltpu.sync_copy(loc_hbm, loc_tab)
    pltpu.sync_copy(scale_hbm, scale_tab)

    def body(j, carry):
        off = pl.multiple_of(j * L, L)
        idx = idx_v[pl.ds(off, L)]
        lv = plsc.load_gather(loc_tab, [idx])
        sv = plsc.load_gather(scale_tab, [idx])
        oloc_v[pl.ds(off, L)] = lv
        oscale_v[pl.ds(off, L)] = _softplus_floor(sv)
        return carry

    lax.fori_loop(0, VECS, body, 0)

    pltpu.sync_copy(oloc_v, out_loc.at[pl.ds(base, BPW)])
    pltpu.sync_copy(oscale_v, out_scale.at[pl.ds(base, BPW)])


def kernel(x, diag_loc, diag_scale):
    loc_t = jnp.pad(diag_loc.reshape(-1), (0, TAB - diag_loc.shape[0]))
    scale_t = jnp.pad(diag_scale.reshape(-1), (0, TAB - diag_scale.shape[0]))
    loc, scale = _gather_softplus(loc_t, scale_t, x.astype(jnp.int32))
    return loc.reshape(-1, 1), scale.reshape(-1, 1)


# speedup vs baseline: 8.1807x; 8.1807x over previous
import functools

import jax
import jax.numpy as jnp
from jax import lax
from jax.experimental import pallas as pl
from jax.experimental.pallas import tpu as pltpu
from jax.experimental.pallas import tpu_sc as plsc

NC = 2
NS = 16
L = 16
NW = NC * NS

B = 16384
TAB = 1024
BPW = B // NW
VECS = BPW // L


def _softplus_floor(s):
    e = jnp.exp(-jnp.abs(s))
    u = e / (e + 2.0)
    t = u * u
    p = 1.0 / 11.0
    p = p * t + 1.0 / 9.0
    p = p * t + 1.0 / 7.0
    p = p * t + 1.0 / 5.0
    p = p * t + 1.0 / 3.0
    p = p * t + 1.0
    log1p_e = 2.0 * u * p
    sp = jnp.maximum(s, 0.0) + log1p_e
    return jnp.maximum(sp, 0.001)


@functools.partial(
    pl.kernel,
    out_type=(
        jax.ShapeDtypeStruct((B,), jnp.float32),
        jax.ShapeDtypeStruct((B,), jnp.float32),
    ),
    mesh=plsc.VectorSubcoreMesh(
        core_axis_name="c", subcore_axis_name="s", num_cores=NC, num_subcores=NS
    ),
    compiler_params=pltpu.CompilerParams(needs_layout_passes=False),
    scratch_types=[
        pltpu.VMEM((TAB,), jnp.float32),
        pltpu.VMEM((TAB,), jnp.float32),
        pltpu.VMEM((BPW,), jnp.int32),
        pltpu.VMEM((BPW,), jnp.float32),
        pltpu.VMEM((BPW,), jnp.float32),
    ],
)
def _gather_softplus(loc_hbm, scale_hbm, idx_hbm, out_loc, out_scale,
                     loc_tab, scale_tab, idx_v, oloc_v, oscale_v):
    wid = lax.axis_index("s") * NC + lax.axis_index("c")
    base = wid * BPW
    pltpu.sync_copy(idx_hbm.at[pl.ds(base, BPW)], idx_v)
    pltpu.sync_copy(loc_hbm, loc_tab)
    pltpu.sync_copy(scale_hbm, scale_tab)

    def body(j, carry):
        off = pl.multiple_of(j * L, L)
        idx = idx_v[pl.ds(off, L)]
        lv = plsc.load_gather(loc_tab, [idx])
        sv = plsc.load_gather(scale_tab, [idx])
        oloc_v[pl.ds(off, L)] = lv
        oscale_v[pl.ds(off, L)] = _softplus_floor(sv)
        return carry

    lax.fori_loop(0, VECS, body, 0)

    pltpu.sync_copy(oloc_v, out_loc.at[pl.ds(base, BPW)])
    pltpu.sync_copy(oscale_v, out_scale.at[pl.ds(base, BPW)])


def kernel(x, diag_loc, diag_scale):
    loc_t = jnp.pad(diag_loc.reshape(-1), (0, TAB - diag_loc.shape[0]))
    scale_t = jnp.pad(diag_scale.reshape(-1), (0, TAB - diag_scale.shape[0]))
    loc, scale = _gather_softplus(loc_t, scale_t, x.astype(jnp.int32))
    return loc.reshape(-1, 1), scale.reshape(-1, 1)
